# baseline (speedup 1.0000x reference)
import functools

import jax
import jax.numpy as jnp
from jax.experimental import pallas as pl
from jax.experimental.pallas import tpu as pltpu

N_NODES = 100000
HIDDEN = 128


def _matmul_relu_kern(x_ref, w_ref, b_ref, o_ref):
    acc = jnp.dot(x_ref[...], w_ref[...], preferred_element_type=jnp.float32)
    o_ref[...] = jnp.maximum(acc + b_ref[...], 0.0)


def _matmul_relu(x, w, b, bm):
    m, k = x.shape
    n = w.shape[1]
    grid = (m // bm,)
    return pl.pallas_call(
        _matmul_relu_kern,
        grid=grid,
        in_specs=[
            pl.BlockSpec((bm, k), lambda i: (i, 0)),
            pl.BlockSpec((k, n), lambda i: (0, 0)),
            pl.BlockSpec((1, n), lambda i: (0, 0)),
        ],
        out_specs=pl.BlockSpec((bm, n), lambda i: (i, 0)),
        out_shape=jax.ShapeDtypeStruct((m, n), jnp.float32),
    )(x, w, b.reshape(1, n))


def _msg_kern(g_ref, bf_ref, wh_ref, wb_ref, o_ref):
    acc = jnp.dot(g_ref[...], wh_ref[...], preferred_element_type=jnp.float32)
    acc += jnp.dot(bf_ref[...], wb_ref[...], preferred_element_type=jnp.float32)
    o_ref[...] = jnp.maximum(acc, 0.0)


def _msg_matmul(gathered, bond, wh, wb, bm):
    m = gathered.shape[0]
    kb = bond.shape[1]
    n = wh.shape[1]
    grid = (m // bm,)
    return pl.pallas_call(
        _msg_kern,
        grid=grid,
        in_specs=[
            pl.BlockSpec((bm, HIDDEN), lambda i: (i, 0)),
            pl.BlockSpec((bm, kb), lambda i: (i, 0)),
            pl.BlockSpec((HIDDEN, n), lambda i: (0, 0)),
            pl.BlockSpec((kb, n), lambda i: (0, 0)),
        ],
        out_specs=pl.BlockSpec((bm, n), lambda i: (i, 0)),
        out_shape=jax.ShapeDtypeStruct((m, n), jnp.float32),
    )(gathered, bond, wh, wb)


def _self_kern(agg_ref, h_ref, w_ref, o_ref):
    acc = jnp.dot(h_ref[...], w_ref[...], preferred_element_type=jnp.float32)
    o_ref[...] = jnp.maximum(acc + agg_ref[...], 0.0)


def _self_update(agg, h, w, bm):
    m = h.shape[0]
    grid = (m // bm,)
    return pl.pallas_call(
        _self_kern,
        grid=grid,
        in_specs=[
            pl.BlockSpec((bm, HIDDEN), lambda i: (i, 0)),
            pl.BlockSpec((bm, HIDDEN), lambda i: (i, 0)),
            pl.BlockSpec((HIDDEN, HIDDEN), lambda i: (0, 0)),
        ],
        out_specs=pl.BlockSpec((bm, HIDDEN), lambda i: (i, 0)),
        out_shape=jax.ShapeDtypeStruct((m, HIDDEN), jnp.float32),
    )(agg, h, w)


def _head_kern(gs_ref, cnt_ref, wro_ref, bro_ref, wout_ref, bout_ref, o_ref):
    g = gs_ref[...] / jnp.maximum(cnt_ref[...], 1.0)
    emb = jnp.maximum(
        jnp.dot(g, wro_ref[...], preferred_element_type=jnp.float32) + bro_ref[...],
        0.0,
    )
    o_ref[...] = jnp.dot(emb, wout_ref[...], preferred_element_type=jnp.float32) + bout_ref[0, 0]


def _head(g_sum, counts, w_ro, b_ro, w_out, b_out):
    ngr = g_sum.shape[0]
    emb_d = w_ro.shape[1]
    out = pl.pallas_call(
        _head_kern,
        in_specs=[
            pl.BlockSpec((ngr, HIDDEN), lambda: (0, 0)),
            pl.BlockSpec((ngr, 1), lambda: (0, 0)),
            pl.BlockSpec((HIDDEN, emb_d), lambda: (0, 0)),
            pl.BlockSpec((1, emb_d), lambda: (0, 0)),
            pl.BlockSpec((emb_d, 1), lambda: (0, 0)),
            pl.BlockSpec((1, 1), lambda: (0, 0)),
        ],
        out_specs=pl.BlockSpec((ngr, 1), lambda: (0, 0)),
        out_shape=jax.ShapeDtypeStruct((ngr, 1), jnp.float32),
    )(g_sum, counts, w_ro, b_ro.reshape(1, emb_d), w_out, b_out.reshape(1, 1))
    return out[:, 0]


def kernel(atom_feature, edge_index, bond_feature, node2graph,
           W_in, b_in, W_msg, W_self, W_ro, b_ro, W_out, b_out):
    src = edge_index[0]
    dst = edge_index[1]
    n_layers = W_msg.shape[0]
    n_graphs = 100

    h = _matmul_relu(atom_feature, W_in, b_in, bm=2000)

    for l in range(n_layers):
        wh = W_msg[l, :HIDDEN, :]
        wb = W_msg[l, HIDDEN:, :]
        gathered = jnp.take(h, src, axis=0)
        msg = _msg_matmul(gathered, bond_feature, wh, wb, bm=2000)
        agg = jax.ops.segment_sum(msg, dst, num_segments=N_NODES)
        h = _self_update(agg, h, W_self[l], bm=2000)

    ones = jnp.ones((N_NODES, 1), dtype=jnp.float32)
    counts = jax.ops.segment_sum(ones, node2graph, num_segments=n_graphs)
    g_sum = jax.ops.segment_sum(h, node2graph, num_segments=n_graphs)
    return _head(g_sum, counts, W_ro, b_ro, W_out, b_out)
